# baseline (speedup 1.0000x reference)
import jax
import jax.numpy as jnp
from jax.experimental import pallas as pl
from jax.experimental.pallas import tpu as pltpu

D_MODEL = 1024
NUM_EXPERTS = 8
N_TOKENS = 4096
TOKEN_BLOCK = 512


def _moe_body(x_ref, gw_ref, gb_ref, ew_ref, eb_ref, out_ref, xs_ref):
    logits = (
        jnp.dot(x_ref[...], gw_ref[...], preferred_element_type=jnp.float32)
        + gb_ref[...]
    )
    iota = jax.lax.broadcasted_iota(jnp.int32, logits.shape, 1)
    m1 = jnp.max(logits, axis=-1, keepdims=True)
    idx1 = jnp.min(
        jnp.where(logits == m1, iota, NUM_EXPERTS), axis=-1, keepdims=True
    )
    one1 = iota == idx1
    masked = jnp.where(one1, -jnp.inf, logits)
    m2 = jnp.max(masked, axis=-1, keepdims=True)
    idx2 = jnp.min(
        jnp.where(masked == m2, iota, NUM_EXPERTS), axis=-1, keepdims=True
    )
    one2 = iota == idx2
    c1 = 1.0 / (1.0 + jnp.exp(m2 - m1))
    cmat = jnp.where(one1, c1, 0.0) + jnp.where(one2, 1.0 - c1, 0.0)

    x = x_ref[...]
    for j in range(NUM_EXPERTS):
        xs_ref[:, j * D_MODEL : (j + 1) * D_MODEL] = x * cmat[:, j : j + 1]

    out_ref[...] = jnp.dot(
        xs_ref[...], ew_ref[...], preferred_element_type=jnp.float32
    ) + jnp.dot(cmat, eb_ref[...], preferred_element_type=jnp.float32)


@jax.jit
def kernel(x, gate_W, gate_b, expert_W, expert_b):
    n_tb = N_TOKENS // TOKEN_BLOCK
    gb2 = gate_b.reshape(1, NUM_EXPERTS)
    ew_all = expert_W.reshape(NUM_EXPERTS * D_MODEL, D_MODEL)
    return pl.pallas_call(
        _moe_body,
        grid=(n_tb,),
        in_specs=[
            pl.BlockSpec((TOKEN_BLOCK, D_MODEL), lambda t: (t, 0)),
            pl.BlockSpec((D_MODEL, NUM_EXPERTS), lambda t: (0, 0)),
            pl.BlockSpec((1, NUM_EXPERTS), lambda t: (0, 0)),
            pl.BlockSpec((NUM_EXPERTS * D_MODEL, D_MODEL), lambda t: (0, 0)),
            pl.BlockSpec((NUM_EXPERTS, D_MODEL), lambda t: (0, 0)),
        ],
        out_specs=pl.BlockSpec((TOKEN_BLOCK, D_MODEL), lambda t: (t, 0)),
        out_shape=jax.ShapeDtypeStruct((N_TOKENS, D_MODEL), jnp.float32),
        scratch_shapes=[
            pltpu.VMEM((TOKEN_BLOCK, NUM_EXPERTS * D_MODEL), jnp.float32),
        ],
    )(x, gate_W, gb2, ew_all, expert_b)
